# baseline (speedup 1.0000x reference)
import functools

import jax
import jax.numpy as jnp
from jax import lax
from jax.experimental import pallas as pl
from jax.experimental.pallas import tpu as pltpu
from jax.experimental.pallas import tpu_sc as plsc

B = 4
N = 10000
K = 16
KL = 8
EMB = 128
HEMB = EMB // 2
BN = B * N

NW = 32
NODES_PW = BN // NW
VR = EMB // 16
HVR = VR // 2

CH = 5
NCH = NODES_PW // CH
IA = CH * KL
IB = CH * K

_MESH = plsc.VectorSubcoreMesh(core_axis_name="c", subcore_axis_name="s")


def _round_bf16_hi(x):
    u = lax.bitcast_convert_type(x, jnp.uint32)
    r = u + jnp.uint32(0x7FFF) + ((u >> jnp.uint32(16)) & jnp.uint32(1))
    return r & jnp.uint32(0xFFFF0000)


def _pack_bf16_pair(a, b):
    w = _round_bf16_hi(a) | (_round_bf16_hi(b) >> jnp.uint32(16))
    return lax.bitcast_convert_type(w, jnp.float32)


def _unpack_bf16_pair(w):
    u = lax.bitcast_convert_type(w, jnp.uint32)
    a = lax.bitcast_convert_type(u & jnp.uint32(0xFFFF0000), jnp.float32)
    b = lax.bitcast_convert_type(u << jnp.uint32(16), jnp.float32)
    return a, b


@functools.partial(
    pl.kernel,
    out_type=(
        jax.ShapeDtypeStruct((BN, HEMB), jnp.float32),
        jax.ShapeDtypeStruct((BN, HEMB), jnp.float32),
        jax.ShapeDtypeStruct((NW * EMB,), jnp.float32),
    ),
    mesh=_MESH,
    compiler_params=pltpu.CompilerParams(use_tc_tiling_on_sc=False),
    scratch_types=(
        pltpu.VMEM((NODES_PW * KL,), jnp.int32),
        pltpu.VMEM((NODES_PW * K,), jnp.int32),
        pltpu.VMEM((IA, HEMB), jnp.float32),
        pltpu.VMEM((IA, HEMB), jnp.float32),
        pltpu.VMEM((CH * EMB,), jnp.float32),
        pltpu.VMEM((CH * EMB,), jnp.float32),
        pltpu.VMEM((CH, HEMB), jnp.float32),
        pltpu.VMEM((CH, HEMB), jnp.float32),
        pltpu.VMEM((CH, HEMB), jnp.float32),
        pltpu.VMEM((CH, HEMB), jnp.float32),
        pltpu.VMEM((IB, HEMB), jnp.float32),
        pltpu.VMEM((IB, HEMB), jnp.float32),
        pltpu.VMEM((CH, HEMB), jnp.float32),
        pltpu.VMEM((CH, HEMB), jnp.float32),
        pltpu.VMEM((EMB,), jnp.float32),
        pltpu.SemaphoreType.DMA,
        pltpu.SemaphoreType.DMA,
        pltpu.SemaphoreType.DMA,
        pltpu.SemaphoreType.DMA,
        pltpu.SemaphoreType.DMA,
        pltpu.SemaphoreType.DMA,
        pltpu.SemaphoreType.DMA,
        pltpu.SemaphoreType.DMA,
        pltpu.SemaphoreType.DMA,
        pltpu.SemaphoreType.DMA,
        pltpu.SemaphoreType.DMA,
        pltpu.SemaphoreType.DMA,
        pltpu.SemaphoreType.DMA,
    ),
)
def _fused(libw_hbm, word_hbm, idxa_hbm, idxb_hbm, s_hbm, h2d, part_hbm,
           idxa_v, idxb_v, ra0, ra1, wv0, wv1, sb0, sb1, hb0, hb1,
           rb0, rb1, sv0, sv1, acc_v,
           sga0, sga1, sw0, sw1, ss0, ss1, sh0, sh1,
           sgb0, sgb1, sr0, sr1, sib):
    cid = lax.axis_index("c")
    sid = lax.axis_index("s")
    batch = cid * 2 + sid // 8
    r = sid % 8
    node_base = batch * N + r * NODES_PW
    prow = r * B + batch

    pltpu.async_copy(idxb_hbm.at[pl.ds(node_base * K, NODES_PW * K)],
                     idxb_v, sib)
    pltpu.sync_copy(idxa_hbm.at[pl.ds(node_base * KL, NODES_PW * KL)], idxa_v)

    abufs = ((ra0, wv0, sb0, hb0, sga0, sw0, ss0, sh0),
             (ra1, wv1, sb1, hb1, sga1, sw1, ss1, sh1))

    def issue_a(c, rows, wv, sg, sw):
        ib = pl.multiple_of(c * IA, 8)
        fb = pl.multiple_of((node_base + c * CH) * EMB, 8)
        pltpu.async_copy(libw_hbm.at[idxa_v.at[pl.ds(ib, IA)]], rows, sg)
        pltpu.async_copy(word_hbm.at[pl.ds(fb, CH * EMB)], wv, sw)

    for bi, bt in enumerate(abufs):
        issue_a(bi, bt[0], bt[1], bt[4], bt[5])

    @pl.loop(0, NCH // 2)
    def _ga(g):
        for bi, (rows, wv, sb, hb, sg, sw, ss, sh) in enumerate(abufs):
            c = g * 2 + bi
            nb = node_base + c * CH
            fb = pl.multiple_of(nb * EMB, 8)

            @pl.when(g > 0)
            def _():
                pltpu.make_async_copy(
                    sb, s_hbm.at[pl.ds(nb, CH)], ss).wait()
                pltpu.make_async_copy(
                    hb, h2d.at[pl.ds(nb, CH)], sh).wait()

            ib = pl.multiple_of(c * IA, 8)
            pltpu.make_async_copy(
                libw_hbm.at[idxa_v.at[pl.ds(ib, IA)]], rows, sg).wait()
            pltpu.make_async_copy(
                word_hbm.at[pl.ds(fb, CH * EMB)], wv, sw).wait()

            @pl.loop(0, CH)
            def _node(i):
                ie = pl.multiple_of(i * EMB, 8)
                t0 = tuple(wv[pl.ds(ie + v * 16, 16)] for v in range(VR))

                @pl.loop(0, KL, init_carry=t0, unroll=4)
                def accs(j, t):
                    out = list(t)
                    for v in range(HVR):
                        w = rows[i * KL + j, pl.ds(v * 16, 16)]
                        e0, e1 = _unpack_bf16_pair(w)
                        out[2 * v] = out[2 * v] + e0
                        out[2 * v + 1] = out[2 * v + 1] + e1
                    return tuple(out)

                for v in range(HVR):
                    sb[i, pl.ds(v * 16, 16)] = _pack_bf16_pair(
                        accs[2 * v], accs[2 * v + 1])
                    ha = jnp.maximum(accs[2 * v], 0.0) * (1.0 / K)
                    hc = jnp.maximum(accs[2 * v + 1], 0.0) * (1.0 / K)
                    hb[i, pl.ds(v * 16, 16)] = _pack_bf16_pair(ha, hc)

            pltpu.async_copy(sb, s_hbm.at[pl.ds(nb, CH)], ss)
            pltpu.async_copy(hb, h2d.at[pl.ds(nb, CH)], sh)

            @pl.when(g < NCH // 2 - 1)
            def _():
                issue_a(c + 2, rows, wv, sg, sw)

    for bi, (_, _, sb, hb, _, _, ss, sh) in enumerate(abufs):
        c = NCH - 2 + bi
        nb = node_base + c * CH
        pltpu.make_async_copy(sb, s_hbm.at[pl.ds(nb, CH)], ss).wait()
        pltpu.make_async_copy(hb, h2d.at[pl.ds(nb, CH)], sh).wait()

    plsc.subcore_barrier()

    pltpu.make_async_copy(
        idxb_hbm.at[pl.ds(node_base * K, NODES_PW * K)], idxb_v, sib).wait()
    for v in range(VR):
        acc_v[pl.ds(v * 16, 16)] = jnp.zeros((16,), jnp.float32)

    bbufs = ((rb0, sv0, sgb0, sr0), (rb1, sv1, sgb1, sr1))

    def issue_b(c, rows, sv, sg, sr):
        ib = pl.multiple_of(c * IB, 8)
        nb = node_base + c * CH
        pltpu.async_copy(h2d.at[idxb_v.at[pl.ds(ib, IB)]], rows, sg)
        pltpu.async_copy(s_hbm.at[pl.ds(nb, CH)], sv, sr)

    for bi, (rows, sv, sg, sr) in enumerate(bbufs):
        issue_b(bi, rows, sv, sg, sr)

    @pl.loop(0, NCH // 2)
    def _gb(g):
        for bi, (rows, sv, sg, sr) in enumerate(bbufs):
            c = g * 2 + bi
            ib = pl.multiple_of(c * IB, 8)
            nb = node_base + c * CH
            pltpu.make_async_copy(
                h2d.at[idxb_v.at[pl.ds(ib, IB)]], rows, sg).wait()
            pltpu.make_async_copy(
                s_hbm.at[pl.ds(nb, CH)], sv, sr).wait()

            accs = [acc_v[pl.ds(v * 16, 16)] for v in range(VR)]
            for i in range(CH):
                sp = [_unpack_bf16_pair(sv[i, pl.ds(u * 16, 16)])
                      for u in range(HVR)]
                t0 = tuple(sp[v // 2][v % 2] for v in range(VR))

                @pl.loop(0, K, init_carry=t0, unroll=4)
                def t(j, tc):
                    out = list(tc)
                    for v in range(HVR):
                        w = rows[i * K + j, pl.ds(v * 16, 16)]
                        e0, e1 = _unpack_bf16_pair(w)
                        out[2 * v] = out[2 * v] + e0
                        out[2 * v + 1] = out[2 * v + 1] + e1
                    return tuple(out)

                for v in range(VR):
                    accs[v] = accs[v] + jnp.maximum(t[v], 0.0)
            for v in range(VR):
                acc_v[pl.ds(v * 16, 16)] = accs[v]

            @pl.when(g < NCH // 2 - 1)
            def _():
                issue_b(c + 2, rows, sv, sg, sr)

    pltpu.sync_copy(acc_v, part_hbm.at[pl.ds(prow * EMB, EMB)])


def _mm_body(x_ref, wh_ref, wl_ref, o_ref):
    x = x_ref[...]
    hi = jnp.dot(x, wh_ref[...], preferred_element_type=jnp.float32)
    lo = jnp.dot(x, wl_ref[...], preferred_element_type=jnp.float32)
    o_ref[...] = _pack_bf16_pair(hi * (1.0 / KL), lo * (1.0 / KL))


def _final_body(p_ref, w2_ref, o_ref):
    p = p_ref[...]
    s = p[0:B] + p[B:2 * B] + p[2 * B:3 * B] + p[3 * B:4 * B]
    s = s + p[4 * B:5 * B] + p[5 * B:6 * B] + p[6 * B:7 * B] + p[7 * B:8 * B]
    o_ref[...] = jnp.dot(s, w2_ref[...], preferred_element_type=jnp.float32)


_MM_BLK = 2000


def kernel(word_embs, neibors, lib_embs, neibors_lib, mask, W, W2):
    del mask
    lib2d = lib_embs.reshape(BN, EMB)
    word1d = word_embs.reshape(BN * EMB)
    offs = (jnp.arange(B, dtype=jnp.int32) * N)[:, None, None]
    idx_a = (neibors_lib.astype(jnp.int32) + offs).reshape(BN * KL)
    idx_b = (neibors.astype(jnp.int32) + offs).reshape(BN * K)

    ci = jnp.arange(HEMB)
    hi_cols = (ci // 16) * 32 + ci % 16
    w_hi = W[:, hi_cols]
    w_lo = W[:, hi_cols + 16]

    libw = pl.pallas_call(
        _mm_body,
        grid=(BN // _MM_BLK,),
        in_specs=[
            pl.BlockSpec((_MM_BLK, EMB), lambda i: (i, 0)),
            pl.BlockSpec((EMB, HEMB), lambda i: (0, 0)),
            pl.BlockSpec((EMB, HEMB), lambda i: (0, 0)),
        ],
        out_specs=pl.BlockSpec((_MM_BLK, HEMB), lambda i: (i, 0)),
        out_shape=jax.ShapeDtypeStruct((BN, HEMB), jnp.float32),
    )(lib2d, w_hi, w_lo)

    _, _, partials = _fused(libw, word1d, idx_a, idx_b)

    out = pl.pallas_call(
        _final_body,
        out_shape=jax.ShapeDtypeStruct((B, EMB), jnp.float32),
    )(partials.reshape(NW, EMB), W2)
    return out


# speedup vs baseline: 1.0024x; 1.0024x over previous
import functools

import jax
import jax.numpy as jnp
from jax import lax
from jax.experimental import pallas as pl
from jax.experimental.pallas import tpu as pltpu
from jax.experimental.pallas import tpu_sc as plsc

B = 4
N = 10000
K = 16
KL = 8
EMB = 128
HEMB = EMB // 2
BN = B * N

NW = 32
NODES_PW = BN // NW
VR = EMB // 16
HVR = VR // 2

CH = 5
NCH = NODES_PW // CH
IA = CH * KL
IB = CH * K

_MESH = plsc.VectorSubcoreMesh(core_axis_name="c", subcore_axis_name="s")


def _round_bf16_hi(x):
    u = lax.bitcast_convert_type(x, jnp.uint32)
    r = u + jnp.uint32(0x7FFF) + ((u >> jnp.uint32(16)) & jnp.uint32(1))
    return r & jnp.uint32(0xFFFF0000)


def _pack_bf16_pair(a, b):
    w = _round_bf16_hi(a) | (_round_bf16_hi(b) >> jnp.uint32(16))
    return lax.bitcast_convert_type(w, jnp.float32)


def _unpack_bf16_pair(w):
    u = lax.bitcast_convert_type(w, jnp.uint32)
    a = lax.bitcast_convert_type(u & jnp.uint32(0xFFFF0000), jnp.float32)
    b = lax.bitcast_convert_type(u << jnp.uint32(16), jnp.float32)
    return a, b


@functools.partial(
    pl.kernel,
    out_type=(
        jax.ShapeDtypeStruct((BN * EMB,), jnp.float32),
        jax.ShapeDtypeStruct((BN, HEMB), jnp.float32),
        jax.ShapeDtypeStruct((NW * EMB,), jnp.float32),
    ),
    mesh=_MESH,
    compiler_params=pltpu.CompilerParams(use_tc_tiling_on_sc=False),
    scratch_types=(
        pltpu.VMEM((NODES_PW * KL,), jnp.int32),
        pltpu.VMEM((NODES_PW * K,), jnp.int32),
        pltpu.VMEM((IA, HEMB), jnp.float32),
        pltpu.VMEM((IA, HEMB), jnp.float32),
        pltpu.VMEM((CH * EMB,), jnp.float32),
        pltpu.VMEM((CH * EMB,), jnp.float32),
        pltpu.VMEM((CH * EMB,), jnp.float32),
        pltpu.VMEM((CH * EMB,), jnp.float32),
        pltpu.VMEM((CH, HEMB), jnp.float32),
        pltpu.VMEM((CH, HEMB), jnp.float32),
        pltpu.VMEM((IB, HEMB), jnp.float32),
        pltpu.VMEM((IB, HEMB), jnp.float32),
        pltpu.VMEM((CH * EMB,), jnp.float32),
        pltpu.VMEM((CH * EMB,), jnp.float32),
        pltpu.VMEM((EMB,), jnp.float32),
        pltpu.SemaphoreType.DMA,
        pltpu.SemaphoreType.DMA,
        pltpu.SemaphoreType.DMA,
        pltpu.SemaphoreType.DMA,
        pltpu.SemaphoreType.DMA,
        pltpu.SemaphoreType.DMA,
        pltpu.SemaphoreType.DMA,
        pltpu.SemaphoreType.DMA,
        pltpu.SemaphoreType.DMA,
        pltpu.SemaphoreType.DMA,
        pltpu.SemaphoreType.DMA,
        pltpu.SemaphoreType.DMA,
        pltpu.SemaphoreType.DMA,
    ),
)
def _fused(libw_hbm, word_hbm, idxa_hbm, idxb_hbm, s_hbm, h2d, part_hbm,
           idxa_v, idxb_v, ra0, ra1, wv0, wv1, sb0, sb1, hb0, hb1,
           rb0, rb1, sv0, sv1, acc_v,
           sga0, sga1, sw0, sw1, ss0, ss1, sh0, sh1,
           sgb0, sgb1, sr0, sr1, sib):
    cid = lax.axis_index("c")
    sid = lax.axis_index("s")
    batch = cid * 2 + sid // 8
    r = sid % 8
    node_base = batch * N + r * NODES_PW
    prow = r * B + batch

    pltpu.async_copy(idxb_hbm.at[pl.ds(node_base * K, NODES_PW * K)],
                     idxb_v, sib)
    pltpu.sync_copy(idxa_hbm.at[pl.ds(node_base * KL, NODES_PW * KL)], idxa_v)

    abufs = ((ra0, wv0, sb0, hb0, sga0, sw0, ss0, sh0),
             (ra1, wv1, sb1, hb1, sga1, sw1, ss1, sh1))

    def issue_a(c, rows, wv, sg, sw):
        ib = pl.multiple_of(c * IA, 8)
        fb = pl.multiple_of((node_base + c * CH) * EMB, 8)
        pltpu.async_copy(libw_hbm.at[idxa_v.at[pl.ds(ib, IA)]], rows, sg)
        pltpu.async_copy(word_hbm.at[pl.ds(fb, CH * EMB)], wv, sw)

    for bi, bt in enumerate(abufs):
        issue_a(bi, bt[0], bt[1], bt[4], bt[5])

    @pl.loop(0, NCH // 2)
    def _ga(g):
        for bi, (rows, wv, sb, hb, sg, sw, ss, sh) in enumerate(abufs):
            c = g * 2 + bi
            nb = node_base + c * CH
            fb = pl.multiple_of(nb * EMB, 8)

            @pl.when(g > 0)
            def _():
                pltpu.make_async_copy(
                    sb, s_hbm.at[pl.ds(fb, CH * EMB)], ss).wait()
                pltpu.make_async_copy(
                    hb, h2d.at[pl.ds(nb, CH)], sh).wait()

            ib = pl.multiple_of(c * IA, 8)
            pltpu.make_async_copy(
                libw_hbm.at[idxa_v.at[pl.ds(ib, IA)]], rows, sg).wait()
            pltpu.make_async_copy(
                word_hbm.at[pl.ds(fb, CH * EMB)], wv, sw).wait()

            @pl.loop(0, CH)
            def _node(i):
                ie = pl.multiple_of(i * EMB, 8)
                t0 = tuple(wv[pl.ds(ie + v * 16, 16)] for v in range(VR))

                @pl.loop(0, KL, init_carry=t0, unroll=4)
                def accs(j, t):
                    out = list(t)
                    for v in range(HVR):
                        w = rows[i * KL + j, pl.ds(v * 16, 16)]
                        e0, e1 = _unpack_bf16_pair(w)
                        out[2 * v] = out[2 * v] + e0
                        out[2 * v + 1] = out[2 * v + 1] + e1
                    return tuple(out)

                for v in range(VR):
                    sb[pl.ds(ie + v * 16, 16)] = accs[v]
                for v in range(HVR):
                    ha = jnp.maximum(accs[2 * v], 0.0) * (1.0 / K)
                    hc = jnp.maximum(accs[2 * v + 1], 0.0) * (1.0 / K)
                    hb[i, pl.ds(v * 16, 16)] = _pack_bf16_pair(ha, hc)

            pltpu.async_copy(sb, s_hbm.at[pl.ds(fb, CH * EMB)], ss)
            pltpu.async_copy(hb, h2d.at[pl.ds(nb, CH)], sh)

            @pl.when(g < NCH // 2 - 1)
            def _():
                issue_a(c + 2, rows, wv, sg, sw)

    for bi, (_, _, sb, hb, _, _, ss, sh) in enumerate(abufs):
        c = NCH - 2 + bi
        nb = node_base + c * CH
        fb = pl.multiple_of(nb * EMB, 8)
        pltpu.make_async_copy(sb, s_hbm.at[pl.ds(fb, CH * EMB)], ss).wait()
        pltpu.make_async_copy(hb, h2d.at[pl.ds(nb, CH)], sh).wait()

    plsc.subcore_barrier()

    pltpu.make_async_copy(
        idxb_hbm.at[pl.ds(node_base * K, NODES_PW * K)], idxb_v, sib).wait()
    for v in range(VR):
        acc_v[pl.ds(v * 16, 16)] = jnp.zeros((16,), jnp.float32)

    bbufs = ((rb0, sv0, sgb0, sr0), (rb1, sv1, sgb1, sr1))

    def issue_b(c, rows, sv, sg, sr):
        ib = pl.multiple_of(c * IB, 8)
        fb = pl.multiple_of((node_base + c * CH) * EMB, 8)
        pltpu.async_copy(h2d.at[idxb_v.at[pl.ds(ib, IB)]], rows, sg)
        pltpu.async_copy(s_hbm.at[pl.ds(fb, CH * EMB)], sv, sr)

    for bi, (rows, sv, sg, sr) in enumerate(bbufs):
        issue_b(bi, rows, sv, sg, sr)

    @pl.loop(0, NCH // 2)
    def _gb(g):
        for bi, (rows, sv, sg, sr) in enumerate(bbufs):
            c = g * 2 + bi
            ib = pl.multiple_of(c * IB, 8)
            fb = pl.multiple_of((node_base + c * CH) * EMB, 8)
            pltpu.make_async_copy(
                h2d.at[idxb_v.at[pl.ds(ib, IB)]], rows, sg).wait()
            pltpu.make_async_copy(
                s_hbm.at[pl.ds(fb, CH * EMB)], sv, sr).wait()

            accs = [acc_v[pl.ds(v * 16, 16)] for v in range(VR)]
            for i in range(CH):
                t0 = tuple(sv[pl.ds(i * EMB + v * 16, 16)] for v in range(VR))

                @pl.loop(0, K, init_carry=t0, unroll=4)
                def t(j, tc):
                    out = list(tc)
                    for v in range(HVR):
                        w = rows[i * K + j, pl.ds(v * 16, 16)]
                        e0, e1 = _unpack_bf16_pair(w)
                        out[2 * v] = out[2 * v] + e0
                        out[2 * v + 1] = out[2 * v + 1] + e1
                    return tuple(out)

                for v in range(VR):
                    accs[v] = accs[v] + jnp.maximum(t[v], 0.0)
            for v in range(VR):
                acc_v[pl.ds(v * 16, 16)] = accs[v]

            @pl.when(g < NCH // 2 - 1)
            def _():
                issue_b(c + 2, rows, sv, sg, sr)

    pltpu.sync_copy(acc_v, part_hbm.at[pl.ds(prow * EMB, EMB)])


def _mm_body(x_ref, wh_ref, wl_ref, o_ref):
    x = x_ref[...]
    hi = jnp.dot(x, wh_ref[...], preferred_element_type=jnp.float32)
    lo = jnp.dot(x, wl_ref[...], preferred_element_type=jnp.float32)
    o_ref[...] = _pack_bf16_pair(hi * (1.0 / KL), lo * (1.0 / KL))


def _final_body(p_ref, w2_ref, o_ref):
    p = p_ref[...]
    s = p[0:B] + p[B:2 * B] + p[2 * B:3 * B] + p[3 * B:4 * B]
    s = s + p[4 * B:5 * B] + p[5 * B:6 * B] + p[6 * B:7 * B] + p[7 * B:8 * B]
    o_ref[...] = jnp.dot(s, w2_ref[...], preferred_element_type=jnp.float32)


_MM_BLK = 2000


def kernel(word_embs, neibors, lib_embs, neibors_lib, mask, W, W2):
    del mask
    lib2d = lib_embs.reshape(BN, EMB)
    word1d = word_embs.reshape(BN * EMB)
    offs = (jnp.arange(B, dtype=jnp.int32) * N)[:, None, None]
    idx_a = (neibors_lib.astype(jnp.int32) + offs).reshape(BN * KL)
    idx_b = (neibors.astype(jnp.int32) + offs).reshape(BN * K)

    ci = jnp.arange(HEMB)
    hi_cols = (ci // 16) * 32 + ci % 16
    w_hi = W[:, hi_cols]
    w_lo = W[:, hi_cols + 16]

    libw = pl.pallas_call(
        _mm_body,
        grid=(BN // _MM_BLK,),
        in_specs=[
            pl.BlockSpec((_MM_BLK, EMB), lambda i: (i, 0)),
            pl.BlockSpec((EMB, HEMB), lambda i: (0, 0)),
            pl.BlockSpec((EMB, HEMB), lambda i: (0, 0)),
        ],
        out_specs=pl.BlockSpec((_MM_BLK, HEMB), lambda i: (i, 0)),
        out_shape=jax.ShapeDtypeStruct((BN, HEMB), jnp.float32),
    )(lib2d, w_hi, w_lo)

    _, _, partials = _fused(libw, word1d, idx_a, idx_b)

    out = pl.pallas_call(
        _final_body,
        out_shape=jax.ShapeDtypeStruct((B, EMB), jnp.float32),
    )(partials.reshape(NW, EMB), W2)
    return out
